# baseline (speedup 1.0000x reference)
import math

import jax
import jax.numpy as jnp
import numpy as np
from jax.experimental import pallas as pl
from jax.experimental.pallas import tpu as pltpu

_N_BOX = 20000
_NPAD = 20480
_C = 8
_T = 100
_NCAND = _C * _T
_IOU_THR = 0.3
_SCORE_THR = 0.7
_SIGMA = 0.5
_CLIP_RATIO = 16.0 / 1000.0


def _nms_body(score_ref, logits_ref, regress_ref, anchors_ref,
              logit_out, boxes_out, cscore_out, cidx_out,
              s_ref, cat_ref, area_ref):
    f32 = jnp.float32
    logitT = score_ref[...] * logits_ref[...]
    logit_out[...] = logitT

    ax1 = anchors_ref[0:1, :]
    ay1 = anchors_ref[1:2, :]
    ax2 = anchors_ref[2:3, :]
    ay2 = anchors_ref[3:4, :]
    ws = ax2 - ax1
    hs = ay2 - ay1
    cx = ax1 + 0.5 * ws
    cy = ay1 + 0.5 * hs
    dx = regress_ref[0:1, :]
    dy = regress_ref[1:2, :]
    mr = f32(abs(math.log(_CLIP_RATIO)))
    dw = jnp.clip(regress_ref[2:3, :], -mr, mr)
    dh = jnp.clip(regress_ref[3:4, :], -mr, mr)
    pcx = cx + dx * ws
    pcy = cy + dy * hs
    pw = ws * jnp.exp(dw)
    ph = hs * jnp.exp(dh)
    x1 = jnp.clip(pcx - 0.5 * pw, 0.0, 1.0)
    y1 = jnp.clip(pcy - 0.5 * ph, 0.0, 1.0)
    x2 = jnp.clip(pcx + 0.5 * pw, 0.0, 1.0)
    y2 = jnp.clip(pcy + 0.5 * ph, 0.0, 1.0)
    boxesT = jnp.concatenate([x1, y1, x2, y2], axis=0)
    boxes_out[...] = boxesT
    cat_ref[...] = jnp.concatenate([boxesT, logitT], axis=0)
    area_ref[...] = jnp.maximum(x2 - x1, 0.0) * jnp.maximum(y2 - y1, 0.0)

    m8 = jnp.max(logitT, axis=0, keepdims=True)
    validm = (m8 >= _SCORE_THR) & (logitT[0:1, :] < m8)
    row = jax.lax.broadcasted_iota(jnp.int32, (_C, _NPAD), 0)
    s0 = jnp.where(validm, logitT, 0.0)
    s0 = jnp.where(row == 0, f32(-1e9), s0)
    s_ref[...] = s0

    lane_f = jax.lax.broadcasted_iota(jnp.int32, (_C, _NPAD), 1).astype(f32)
    row8 = jax.lax.broadcasted_iota(jnp.int32, (_C, 1), 0)
    eye8 = (jax.lax.broadcasted_iota(jnp.int32, (_C, _C), 0)
            == jax.lax.broadcasted_iota(jnp.int32, (_C, _C), 1)).astype(f32)
    col128 = jax.lax.broadcasted_iota(jnp.int32, (1, 128), 1)
    cscore_out[...] = jnp.zeros((_C, 128), f32)
    cidx_out[...] = jnp.zeros((_C, 128), f32)

    def step(t, carry):
        s = s_ref[...]
        m = jnp.max(s, axis=1, keepdims=True)
        i_f = jnp.min(jnp.where(s == m, lane_f, f32(_NPAD)),
                      axis=1, keepdims=True)
        oh_b = lane_f == i_f
        oh_f = oh_b.astype(f32)
        g = jax.lax.dot_general(oh_f, cat_ref[...],
                                (((1,), (1,)), ((), ())),
                                preferred_element_type=f32,
                                precision=jax.lax.Precision.HIGHEST)
        bx1 = g[:, 0:1]
        by1 = g[:, 1:2]
        bx2 = g[:, 2:3]
        by2 = g[:, 3:4]
        cls = jnp.sum(g[:, 4:12] * eye8, axis=1, keepdims=True)
        cls = jnp.where(row8 == 0, f32(-1e9), cls)
        ix1 = jnp.maximum(bx1, cat_ref[0:1, :])
        iy1 = jnp.maximum(by1, cat_ref[1:2, :])
        ix2 = jnp.minimum(bx2, cat_ref[2:3, :])
        iy2 = jnp.minimum(by2, cat_ref[3:4, :])
        inter = jnp.maximum(ix2 - ix1, 0.0) * jnp.maximum(iy2 - iy1, 0.0)
        a0 = jnp.maximum(bx2 - bx1, 0.0) * jnp.maximum(by2 - by1, 0.0)
        union = a0 + area_ref[...] - inter
        iou = inter / jnp.maximum(union, f32(1e-9))
        w = jnp.where(iou <= _IOU_THR,
                      jnp.exp(-0.5 * iou * iou / f32(_SIGMA)), 0.0)
        s_ref[...] = jnp.where(oh_b, f32(-1.0), s * w)
        oh_t = (col128 == t).astype(f32)
        cscore_out[...] = cscore_out[...] + cls * oh_t
        cidx_out[...] = cidx_out[...] + i_f * oh_t
        return carry

    jax.lax.fori_loop(0, _T, step, 0)


def _topk_body(vcol_ref, vrow_ref, idx_ref, logit_ref, boxes_ref,
               ol_ref, op_ref):
    f32 = jnp.float32
    vcol = vcol_ref[...]
    vrow = vrow_ref[...]
    ri = jax.lax.broadcasted_iota(jnp.int32, (_NCAND, _NCAND), 0)
    ci = jax.lax.broadcasted_iota(jnp.int32, (_NCAND, _NCAND), 1)
    beats = (vcol > vrow) | ((vcol == vrow) & (ri < ci))
    rank = jnp.sum(beats.astype(f32), axis=0, keepdims=True)
    rr = jax.lax.broadcasted_iota(jnp.int32, (_T, _NCAND), 0).astype(f32)
    pr = (rr == rank).astype(f32)
    sel = jax.lax.dot_general(pr, idx_ref[...],
                              (((1,), (0,)), ((), ())),
                              preferred_element_type=f32,
                                precision=jax.lax.Precision.HIGHEST)
    sel_i = sel.astype(jnp.int32)
    li = jax.lax.broadcasted_iota(jnp.int32, (_T, _NPAD), 1)
    oh = (li == sel_i).astype(f32)
    ol_ref[...] = jax.lax.dot_general(oh, logit_ref[...],
                                      (((1,), (1,)), ((), ())),
                                      preferred_element_type=f32,
                                precision=jax.lax.Precision.HIGHEST)
    op_ref[...] = jax.lax.dot_general(oh, boxes_ref[...],
                                      (((1,), (1,)), ((), ())),
                                      preferred_element_type=f32,
                                precision=jax.lax.Precision.HIGHEST)


def _stage1(scoreT, logitsT, regressT, anchorsT):
    f32 = jnp.float32
    return pl.pallas_call(
        _nms_body,
        out_shape=[
            jax.ShapeDtypeStruct((_C, _NPAD), f32),
            jax.ShapeDtypeStruct((4, _NPAD), f32),
            jax.ShapeDtypeStruct((_C, 128), f32),
            jax.ShapeDtypeStruct((_C, 128), f32),
        ],
        scratch_shapes=[
            pltpu.VMEM((_C, _NPAD), f32),
            pltpu.VMEM((12, _NPAD), f32),
            pltpu.VMEM((1, _NPAD), f32),
        ],
    )(scoreT, logitsT, regressT, anchorsT)


def _stage2(v, ix, logitT, boxesT):
    f32 = jnp.float32
    return pl.pallas_call(
        _topk_body,
        out_shape=[
            jax.ShapeDtypeStruct((_T, _C), f32),
            jax.ShapeDtypeStruct((_T, 4), f32),
        ],
    )(v[:, None], v[None, :], ix[:, None], logitT, boxesT)


def _prep_body(score_ref, logits_ref, regress_ref, anchors_ref,
               logit_out, boxes_out, s0_out, area_out):
    f32 = jnp.float32
    logitT = score_ref[...] * logits_ref[...]
    logit_out[...] = logitT
    ax1 = anchors_ref[0:1, :]
    ay1 = anchors_ref[1:2, :]
    ax2 = anchors_ref[2:3, :]
    ay2 = anchors_ref[3:4, :]
    ws = ax2 - ax1
    hs = ay2 - ay1
    cx = ax1 + 0.5 * ws
    cy = ay1 + 0.5 * hs
    dx = regress_ref[0:1, :]
    dy = regress_ref[1:2, :]
    mr = f32(abs(math.log(_CLIP_RATIO)))
    dw = jnp.clip(regress_ref[2:3, :], -mr, mr)
    dh = jnp.clip(regress_ref[3:4, :], -mr, mr)
    pcx = cx + dx * ws
    pcy = cy + dy * hs
    pw = ws * jnp.exp(dw)
    ph = hs * jnp.exp(dh)
    x1 = jnp.clip(pcx - 0.5 * pw, 0.0, 1.0)
    y1 = jnp.clip(pcy - 0.5 * ph, 0.0, 1.0)
    x2 = jnp.clip(pcx + 0.5 * pw, 0.0, 1.0)
    y2 = jnp.clip(pcy + 0.5 * ph, 0.0, 1.0)
    boxes_out[...] = jnp.concatenate([x1, y1, x2, y2], axis=0)
    area_out[...] = jnp.maximum(x2 - x1, 0.0) * jnp.maximum(y2 - y1, 0.0)
    m8 = jnp.max(logitT, axis=0, keepdims=True)
    validm = (m8 >= _SCORE_THR) & (logitT[0:1, :] < m8)
    row = jax.lax.broadcasted_iota(jnp.int32, (_C, _NPAD), 0)
    s0 = jnp.where(validm, logitT, 0.0)
    s0_out[...] = jnp.where(row == 0, f32(-1e9), s0)


def _prep(scoreT, logitsT, regressT, anchorsT):
    f32 = jnp.float32
    return pl.pallas_call(
        _prep_body,
        out_shape=[
            jax.ShapeDtypeStruct((_C, _NPAD), f32),
            jax.ShapeDtypeStruct((4, _NPAD), f32),
            jax.ShapeDtypeStruct((_C, _NPAD), f32),
            jax.ShapeDtypeStruct((1, _NPAD), f32),
        ],
    )(scoreT, logitsT, regressT, anchorsT)


def kernel(score, logits, regress, anchors):
    f32 = jnp.float32
    pad = _NPAD - _N_BOX
    scoreT = jnp.pad(score[0, :, 0][None, :].astype(f32), ((0, 0), (0, pad)))
    logitsT = jnp.pad(logits[0].T.astype(f32), ((0, 0), (0, pad)))
    regressT = jnp.pad(regress[0].T.astype(f32), ((0, 0), (0, pad)))
    anchorsT = jnp.pad(anchors.T.astype(f32), ((0, 0), (0, pad)))
    logitT, boxesT, s0, area = _prep(scoreT, logitsT, regressT, anchorsT)
    cval, cidx = sc_nms(s0, boxesT, area.reshape(-1), logitT)
    cval = cval.at[0].set(f32(-1e9))
    v = cval[:, :_T].reshape(-1)
    ix = cidx[:, :_T].reshape(-1)
    out_logit, out_prop = _stage2(v, ix, logitT, boxesT)
    return out_logit[None], out_prop[None]


from jax import lax
from jax.experimental.pallas import tpu_sc as plsc

_NS = 16
_L = 16
_NPB = _NPAD // _NS
_NCH = _NPB // _L
_KSLOT = 4
_T = 100
_IOU_THR = 0.3
_SIGMA = 0.5
_BIGI = 1e9

def _sc_mesh():
    return plsc.VectorSubcoreMesh(core_axis_name="c", subcore_axis_name="s",
                                  num_cores=2, num_subcores=16)


def _sc_nms_body(s0_hbm, boxes_hbm, area_hbm, logit_hbm,
                 cval_out, cidx_out,
                 s_ref, lg_ref, bx_ref, ar_ref, comb_ref, st_ref, tb_ref,
                 fg_ref, table_sh, acc_sh):
    f32 = jnp.float32
    i32 = jnp.int32
    core = lax.axis_index("c")
    tile = lax.axis_index("s")
    base = tile * _NPB
    lane = lax.broadcasted_iota(i32, (_L,), 0)
    lanef = lane.astype(f32)
    z16 = jnp.zeros((_L,), f32)

    cls_ids = [jnp.where(core == 0, k + 1, k + 5 if k < 3 else 0) for k in range(_KSLOT)]

    for k in range(_KSLOT):
        pltpu.sync_copy(s0_hbm.at[cls_ids[k], pl.ds(base, _NPB)], s_ref.at[k])
        pltpu.sync_copy(logit_hbm.at[cls_ids[k], pl.ds(base, _NPB)], lg_ref.at[k])
    for r in range(4):
        pltpu.sync_copy(boxes_hbm.at[r, pl.ds(base, _NPB)], bx_ref.at[r])
    pltpu.sync_copy(area_hbm.at[pl.ds(base, _NPB)], ar_ref)

    for r in range(16):
        for c in range(8):
            comb_ref[r, pl.ds(c * _L, _L)] = z16

    def step(t, carry):
        winners = []
        for k in range(_KSLOT):
            def cb(j, mvmi, k=k):
                mv, mi = mvmi
                x = s_ref[k, pl.ds(j * _L, _L)]
                gi = ((base + j * _L) + lane).astype(f32)
                cond = x > mv
                return (jnp.where(cond, x, mv), jnp.where(cond, gi, mi))
            mv, mi = lax.fori_loop(0, _NCH, cb,
                                   (jnp.full((_L,), -3e38, f32), z16))
            m = jnp.max(mv)
            il = jnp.min(jnp.where(mv == m, mi, f32(_BIGI)))
            offc = jnp.clip(il.astype(i32) - base, 0, _NPB - 1)
            ov = jnp.full((_L,), offc, i32)
            bx1v = plsc.load_gather(bx_ref, [jnp.full((_L,), 0, i32), ov])
            by1v = plsc.load_gather(bx_ref, [jnp.full((_L,), 1, i32), ov])
            bx2v = plsc.load_gather(bx_ref, [jnp.full((_L,), 2, i32), ov])
            by2v = plsc.load_gather(bx_ref, [jnp.full((_L,), 3, i32), ov])
            r0 = jnp.where(lane == 0, m, 0.0)
            r0 = jnp.where(lane == 1, il, r0)
            r0 = jnp.where(lane == 2, bx1v, r0)
            r0 = jnp.where(lane == 3, by1v, r0)
            r0 = jnp.where(lane == 4, bx2v, r0)
            r0 = jnp.where(lane == 5, by2v, r0)
            st_ref[...] = r0
            pltpu.sync_copy(st_ref, table_sh.at[k, tile])
        plsc.subcore_barrier()

        for k in range(_KSLOT):
            pltpu.sync_copy(table_sh.at[k], tb_ref)
            vals = plsc.load_gather(tb_ref, [lane, jnp.full((_L,), 0, i32)])
            idxs = plsc.load_gather(tb_ref, [lane, jnp.full((_L,), 1, i32)])
            mg = jnp.max(vals)
            iw = jnp.min(jnp.where(vals == mg, idxs, f32(_BIGI)))
            rsel = jnp.min(jnp.where((vals == mg) & (idxs == iw), lanef,
                                     f32(99.0))).astype(i32)
            rv = jnp.full((_L,), rsel, i32)
            bx1w = plsc.load_gather(tb_ref, [rv, jnp.full((_L,), 2, i32)])
            by1w = plsc.load_gather(tb_ref, [rv, jnp.full((_L,), 3, i32)])
            bx2w = plsc.load_gather(tb_ref, [rv, jnp.full((_L,), 4, i32)])
            by2w = plsc.load_gather(tb_ref, [rv, jnp.full((_L,), 5, i32)])
            iwi = iw.astype(i32)
            owner = (iwi >= base) & (iwi < base + _NPB)
            offw = jnp.clip(iwi - base, 0, _NPB - 1)
            clsv = plsc.load_gather(
                lg_ref, [jnp.full((_L,), k, i32), jnp.full((_L,), offw, i32)])
            msk = (lane == 0) & owner
            tvec = jnp.full((_L,), t, i32)
            plsc.store_scatter(comb_ref.at[k], [tvec], clsv, mask=msk)
            plsc.store_scatter(comb_ref.at[4 + k], [tvec],
                               jnp.full((_L,), iw, f32), mask=msk)

            a0 = (jnp.maximum(bx2w - bx1w, 0.0) * jnp.maximum(by2w - by1w, 0.0))
            def db(j, _, k=k, bx1w=bx1w, by1w=by1w, bx2w=bx2w, by2w=by2w,
                   a0=a0, iwi=iwi):
                x = s_ref[k, pl.ds(j * _L, _L)]
                ix1 = jnp.maximum(bx1w, bx_ref[0, pl.ds(j * _L, _L)])
                iy1 = jnp.maximum(by1w, bx_ref[1, pl.ds(j * _L, _L)])
                ix2 = jnp.minimum(bx2w, bx_ref[2, pl.ds(j * _L, _L)])
                iy2 = jnp.minimum(by2w, bx_ref[3, pl.ds(j * _L, _L)])
                inter = (jnp.maximum(ix2 - ix1, 0.0)
                         * jnp.maximum(iy2 - iy1, 0.0))
                union = a0 + ar_ref[pl.ds(j * _L, _L)] - inter
                iou = inter / jnp.maximum(union, jnp.float32(1e-9))
                w = jnp.where(iou <= _IOU_THR,
                              jnp.exp(-0.5 * iou * iou / jnp.float32(_SIGMA)),
                              0.0)
                gi = (base + j * _L) + lane
                s_ref[k, pl.ds(j * _L, _L)] = jnp.where(gi == iwi,
                                                        jnp.float32(-1.0),
                                                        x * w)
                return 0
            lax.fori_loop(0, _NCH, db, 0)
        plsc.subcore_barrier()
        return carry

    lax.fori_loop(0, _T, step, 0)

    pltpu.sync_copy(comb_ref, acc_sh.at[tile])
    plsc.subcore_barrier()
    @pl.when(tile == 0)
    def _():
        def acc_body(tt, carry):
            pltpu.sync_copy(acc_sh.at[tt], fg_ref)
            for r in range(16):
                for c in range(8):
                    sl = pl.ds(c * _L, _L)
                    comb_ref[r, sl] = comb_ref[r, sl] + fg_ref[r, sl]
            return carry
        lax.fori_loop(1, _NS, acc_body, 0)
        for k in range(_KSLOT):
            pltpu.sync_copy(comb_ref.at[k], cval_out.at[cls_ids[k]])
            pltpu.sync_copy(comb_ref.at[4 + k], cidx_out.at[cls_ids[k]])


def sc_nms(s0, boxesT, area, logitT):
    f32 = jnp.float32
    return pl.kernel(
        _sc_nms_body,
        out_type=(jax.ShapeDtypeStruct((8, 128), f32),
                  jax.ShapeDtypeStruct((8, 128), f32)),
        mesh=_sc_mesh(),
        compiler_params=pltpu.CompilerParams(needs_layout_passes=False),
        scratch_types=[
            pltpu.VMEM((_KSLOT, _NPB), f32),
            pltpu.VMEM((_KSLOT, _NPB), f32),
            pltpu.VMEM((4, _NPB), f32),
            pltpu.VMEM((_NPB,), f32),
            pltpu.VMEM((16, 128), f32),
            pltpu.VMEM((_L,), f32),
            pltpu.VMEM((_NS, _L), f32),
            pltpu.VMEM((16, 128), f32),
            pltpu.VMEM_SHARED((_KSLOT, _NS, _L), f32),
            pltpu.VMEM_SHARED((_NS, 16, 128), f32),
        ],
    )(s0, boxesT, area, logitT)


# speedup vs baseline: 1.0542x; 1.0542x over previous
import math

import jax
import jax.numpy as jnp
import numpy as np
from jax.experimental import pallas as pl
from jax.experimental.pallas import tpu as pltpu

_N_BOX = 20000
_NPAD = 20480
_C = 8
_T = 100
_NCAND = _C * _T
_IOU_THR = 0.3
_SCORE_THR = 0.7
_SIGMA = 0.5
_CLIP_RATIO = 16.0 / 1000.0


def _nms_body(score_ref, logits_ref, regress_ref, anchors_ref,
              logit_out, boxes_out, cscore_out, cidx_out,
              s_ref, cat_ref, area_ref):
    f32 = jnp.float32
    logitT = score_ref[...] * logits_ref[...]
    logit_out[...] = logitT

    ax1 = anchors_ref[0:1, :]
    ay1 = anchors_ref[1:2, :]
    ax2 = anchors_ref[2:3, :]
    ay2 = anchors_ref[3:4, :]
    ws = ax2 - ax1
    hs = ay2 - ay1
    cx = ax1 + 0.5 * ws
    cy = ay1 + 0.5 * hs
    dx = regress_ref[0:1, :]
    dy = regress_ref[1:2, :]
    mr = f32(abs(math.log(_CLIP_RATIO)))
    dw = jnp.clip(regress_ref[2:3, :], -mr, mr)
    dh = jnp.clip(regress_ref[3:4, :], -mr, mr)
    pcx = cx + dx * ws
    pcy = cy + dy * hs
    pw = ws * jnp.exp(dw)
    ph = hs * jnp.exp(dh)
    x1 = jnp.clip(pcx - 0.5 * pw, 0.0, 1.0)
    y1 = jnp.clip(pcy - 0.5 * ph, 0.0, 1.0)
    x2 = jnp.clip(pcx + 0.5 * pw, 0.0, 1.0)
    y2 = jnp.clip(pcy + 0.5 * ph, 0.0, 1.0)
    boxesT = jnp.concatenate([x1, y1, x2, y2], axis=0)
    boxes_out[...] = boxesT
    cat_ref[...] = jnp.concatenate([boxesT, logitT], axis=0)
    area_ref[...] = jnp.maximum(x2 - x1, 0.0) * jnp.maximum(y2 - y1, 0.0)

    m8 = jnp.max(logitT, axis=0, keepdims=True)
    validm = (m8 >= _SCORE_THR) & (logitT[0:1, :] < m8)
    row = jax.lax.broadcasted_iota(jnp.int32, (_C, _NPAD), 0)
    s0 = jnp.where(validm, logitT, 0.0)
    s0 = jnp.where(row == 0, f32(-1e9), s0)
    s_ref[...] = s0

    lane_f = jax.lax.broadcasted_iota(jnp.int32, (_C, _NPAD), 1).astype(f32)
    row8 = jax.lax.broadcasted_iota(jnp.int32, (_C, 1), 0)
    eye8 = (jax.lax.broadcasted_iota(jnp.int32, (_C, _C), 0)
            == jax.lax.broadcasted_iota(jnp.int32, (_C, _C), 1)).astype(f32)
    col128 = jax.lax.broadcasted_iota(jnp.int32, (1, 128), 1)
    cscore_out[...] = jnp.zeros((_C, 128), f32)
    cidx_out[...] = jnp.zeros((_C, 128), f32)

    def step(t, carry):
        s = s_ref[...]
        m = jnp.max(s, axis=1, keepdims=True)
        i_f = jnp.min(jnp.where(s == m, lane_f, f32(_NPAD)),
                      axis=1, keepdims=True)
        oh_b = lane_f == i_f
        oh_f = oh_b.astype(f32)
        g = jax.lax.dot_general(oh_f, cat_ref[...],
                                (((1,), (1,)), ((), ())),
                                preferred_element_type=f32,
                                precision=jax.lax.Precision.HIGHEST)
        bx1 = g[:, 0:1]
        by1 = g[:, 1:2]
        bx2 = g[:, 2:3]
        by2 = g[:, 3:4]
        cls = jnp.sum(g[:, 4:12] * eye8, axis=1, keepdims=True)
        cls = jnp.where(row8 == 0, f32(-1e9), cls)
        ix1 = jnp.maximum(bx1, cat_ref[0:1, :])
        iy1 = jnp.maximum(by1, cat_ref[1:2, :])
        ix2 = jnp.minimum(bx2, cat_ref[2:3, :])
        iy2 = jnp.minimum(by2, cat_ref[3:4, :])
        inter = jnp.maximum(ix2 - ix1, 0.0) * jnp.maximum(iy2 - iy1, 0.0)
        a0 = jnp.maximum(bx2 - bx1, 0.0) * jnp.maximum(by2 - by1, 0.0)
        union = a0 + area_ref[...] - inter
        iou = inter / jnp.maximum(union, f32(1e-9))
        w = jnp.where(iou <= _IOU_THR,
                      jnp.exp(-0.5 * iou * iou / f32(_SIGMA)), 0.0)
        s_ref[...] = jnp.where(oh_b, f32(-1.0), s * w)
        oh_t = (col128 == t).astype(f32)
        cscore_out[...] = cscore_out[...] + cls * oh_t
        cidx_out[...] = cidx_out[...] + i_f * oh_t
        return carry

    jax.lax.fori_loop(0, _T, step, 0)


def _topk_body(vcol_ref, vrow_ref, idx_ref, logit_ref, boxes_ref,
               ol_ref, op_ref):
    f32 = jnp.float32
    vcol = vcol_ref[...]
    vrow = vrow_ref[...]
    ri = jax.lax.broadcasted_iota(jnp.int32, (_NCAND, _NCAND), 0)
    ci = jax.lax.broadcasted_iota(jnp.int32, (_NCAND, _NCAND), 1)
    beats = (vcol > vrow) | ((vcol == vrow) & (ri < ci))
    rank = jnp.sum(beats.astype(f32), axis=0, keepdims=True)
    rr = jax.lax.broadcasted_iota(jnp.int32, (_T, _NCAND), 0).astype(f32)
    pr = (rr == rank).astype(f32)
    sel = jax.lax.dot_general(pr, idx_ref[...],
                              (((1,), (0,)), ((), ())),
                              preferred_element_type=f32,
                                precision=jax.lax.Precision.HIGHEST)
    sel_i = sel.astype(jnp.int32)
    li = jax.lax.broadcasted_iota(jnp.int32, (_T, _NPAD), 1)
    oh = (li == sel_i).astype(f32)
    ol_ref[...] = jax.lax.dot_general(oh, logit_ref[...],
                                      (((1,), (1,)), ((), ())),
                                      preferred_element_type=f32,
                                precision=jax.lax.Precision.HIGHEST)
    op_ref[...] = jax.lax.dot_general(oh, boxes_ref[...],
                                      (((1,), (1,)), ((), ())),
                                      preferred_element_type=f32,
                                precision=jax.lax.Precision.HIGHEST)


def _stage1(scoreT, logitsT, regressT, anchorsT):
    f32 = jnp.float32
    return pl.pallas_call(
        _nms_body,
        out_shape=[
            jax.ShapeDtypeStruct((_C, _NPAD), f32),
            jax.ShapeDtypeStruct((4, _NPAD), f32),
            jax.ShapeDtypeStruct((_C, 128), f32),
            jax.ShapeDtypeStruct((_C, 128), f32),
        ],
        scratch_shapes=[
            pltpu.VMEM((_C, _NPAD), f32),
            pltpu.VMEM((12, _NPAD), f32),
            pltpu.VMEM((1, _NPAD), f32),
        ],
    )(scoreT, logitsT, regressT, anchorsT)


def _stage2(v, ix, logitT, boxesT):
    f32 = jnp.float32
    return pl.pallas_call(
        _topk_body,
        out_shape=[
            jax.ShapeDtypeStruct((_T, _C), f32),
            jax.ShapeDtypeStruct((_T, 4), f32),
        ],
    )(v[:, None], v[None, :], ix[:, None], logitT, boxesT)


def _prep_body(score_ref, logits_ref, regress_ref, anchors_ref,
               logit_out, boxes_out, s0_out, area_out):
    f32 = jnp.float32
    logitT = score_ref[...] * logits_ref[...]
    logit_out[...] = logitT
    ax1 = anchors_ref[0:1, :]
    ay1 = anchors_ref[1:2, :]
    ax2 = anchors_ref[2:3, :]
    ay2 = anchors_ref[3:4, :]
    ws = ax2 - ax1
    hs = ay2 - ay1
    cx = ax1 + 0.5 * ws
    cy = ay1 + 0.5 * hs
    dx = regress_ref[0:1, :]
    dy = regress_ref[1:2, :]
    mr = f32(abs(math.log(_CLIP_RATIO)))
    dw = jnp.clip(regress_ref[2:3, :], -mr, mr)
    dh = jnp.clip(regress_ref[3:4, :], -mr, mr)
    pcx = cx + dx * ws
    pcy = cy + dy * hs
    pw = ws * jnp.exp(dw)
    ph = hs * jnp.exp(dh)
    x1 = jnp.clip(pcx - 0.5 * pw, 0.0, 1.0)
    y1 = jnp.clip(pcy - 0.5 * ph, 0.0, 1.0)
    x2 = jnp.clip(pcx + 0.5 * pw, 0.0, 1.0)
    y2 = jnp.clip(pcy + 0.5 * ph, 0.0, 1.0)
    boxes_out[...] = jnp.concatenate([x1, y1, x2, y2], axis=0)
    area_out[...] = jnp.maximum(x2 - x1, 0.0) * jnp.maximum(y2 - y1, 0.0)
    m8 = jnp.max(logitT, axis=0, keepdims=True)
    validm = (m8 >= _SCORE_THR) & (logitT[0:1, :] < m8)
    row = jax.lax.broadcasted_iota(jnp.int32, (_C, _NPAD), 0)
    s0 = jnp.where(validm, logitT, 0.0)
    s0_out[...] = jnp.where(row == 0, f32(-1e9), s0)


def _prep(scoreT, logitsT, regressT, anchorsT):
    f32 = jnp.float32
    return pl.pallas_call(
        _prep_body,
        out_shape=[
            jax.ShapeDtypeStruct((_C, _NPAD), f32),
            jax.ShapeDtypeStruct((4, _NPAD), f32),
            jax.ShapeDtypeStruct((_C, _NPAD), f32),
            jax.ShapeDtypeStruct((1, _NPAD), f32),
        ],
    )(scoreT, logitsT, regressT, anchorsT)


def kernel(score, logits, regress, anchors):
    f32 = jnp.float32
    pad = _NPAD - _N_BOX
    scoreT = jnp.pad(score[0, :, 0][None, :].astype(f32), ((0, 0), (0, pad)))
    logitsT = jnp.pad(logits[0].T.astype(f32), ((0, 0), (0, pad)))
    regressT = jnp.pad(regress[0].T.astype(f32), ((0, 0), (0, pad)))
    anchorsT = jnp.pad(anchors.T.astype(f32), ((0, 0), (0, pad)))
    logitT, boxesT, s0, area = _prep(scoreT, logitsT, regressT, anchorsT)
    cval, cidx = sc_nms(s0, boxesT, area.reshape(-1), logitT)
    cval = cval.at[0].set(f32(-1e9))
    v = cval[:, :_T].reshape(-1)
    ix = cidx[:, :_T].reshape(-1)
    out_logit, out_prop = _stage2(v, ix, logitT, boxesT)
    return out_logit[None], out_prop[None]


from jax import lax
from jax.experimental.pallas import tpu_sc as plsc

_NS = 16
_L = 16
_NPB = _NPAD // _NS
_NCH = _NPB // _L
_KSLOT = 4
_T = 100
_IOU_THR = 0.3
_SIGMA = 0.5
_BIGI = 1e9

def _sc_mesh():
    return plsc.VectorSubcoreMesh(core_axis_name="c", subcore_axis_name="s",
                                  num_cores=2, num_subcores=16)


def _sc_nms_body(s0_hbm, boxes_hbm, area_hbm, logit_hbm,
                 cval_out, cidx_out,
                 s_ref, lg_ref, bx_ref, ar_ref, comb_ref, st_ref, tb_ref,
                 fg_ref, table_sh, acc_sh):
    f32 = jnp.float32
    i32 = jnp.int32
    core = lax.axis_index("c")
    tile = lax.axis_index("s")
    base = tile * _NPB
    lane = lax.broadcasted_iota(i32, (_L,), 0)
    lanef = lane.astype(f32)
    z16 = jnp.zeros((_L,), f32)

    cls_ids = [jnp.where(core == 0, k + 1, k + 5 if k < 3 else 0) for k in range(_KSLOT)]

    for k in range(_KSLOT):
        pltpu.sync_copy(s0_hbm.at[cls_ids[k], pl.ds(base, _NPB)], s_ref.at[k])
        pltpu.sync_copy(logit_hbm.at[cls_ids[k], pl.ds(base, _NPB)], lg_ref.at[k])
    for r in range(4):
        pltpu.sync_copy(boxes_hbm.at[r, pl.ds(base, _NPB)], bx_ref.at[r])
    pltpu.sync_copy(area_hbm.at[pl.ds(base, _NPB)], ar_ref)

    for r in range(16):
        for c in range(8):
            comb_ref[r, pl.ds(c * _L, _L)] = z16

    def _emit_row(k, m, il):
        offc = jnp.clip(il.astype(i32) - base, 0, _NPB - 1)
        ov = jnp.full((_L,), offc, i32)
        bx1v = plsc.load_gather(bx_ref, [jnp.full((_L,), 0, i32), ov])
        by1v = plsc.load_gather(bx_ref, [jnp.full((_L,), 1, i32), ov])
        bx2v = plsc.load_gather(bx_ref, [jnp.full((_L,), 2, i32), ov])
        by2v = plsc.load_gather(bx_ref, [jnp.full((_L,), 3, i32), ov])
        r0 = jnp.where(lane == 0, m, 0.0)
        r0 = jnp.where(lane == 1, il, r0)
        r0 = jnp.where(lane == 2, bx1v, r0)
        r0 = jnp.where(lane == 3, by1v, r0)
        r0 = jnp.where(lane == 4, bx2v, r0)
        r0 = jnp.where(lane == 5, by2v, r0)
        st_ref[k, pl.ds(0, _L)] = r0

    for k in range(_KSLOT):
        def cb(jj, mvmi, k=k):
            mv, mi = mvmi
            for u in range(4):
                j = jj * 4 + u
                x = s_ref[k, pl.ds(j * _L, _L)]
                gi = ((base + j * _L) + lane).astype(f32)
                cond = x > mv
                mv = jnp.where(cond, x, mv)
                mi = jnp.where(cond, gi, mi)
            return (mv, mi)
        mv, mi = lax.fori_loop(0, _NCH // 4, cb,
                               (jnp.full((_L,), -3e38, f32), z16))
        m = jnp.max(mv)
        il = jnp.min(jnp.where(mv == m, mi, f32(_BIGI)))
        _emit_row(k, m, il)
    pltpu.sync_copy(st_ref, table_sh.at[0, tile])
    plsc.subcore_barrier()

    def step(t, carry):
        p = lax.rem(t, 2)
        pltpu.sync_copy(table_sh.at[p], tb_ref)
        for k in range(_KSLOT):
            fullk = jnp.full((_L,), k, i32)
            vals = plsc.load_gather(tb_ref, [lane, fullk,
                                             jnp.full((_L,), 0, i32)])
            idxs = plsc.load_gather(tb_ref, [lane, fullk,
                                             jnp.full((_L,), 1, i32)])
            mg = jnp.max(vals)
            iw = jnp.min(jnp.where(vals == mg, idxs, f32(_BIGI)))
            rsel = jnp.min(jnp.where((vals == mg) & (idxs == iw), lanef,
                                     f32(99.0))).astype(i32)
            rv = jnp.full((_L,), rsel, i32)
            bx1w = plsc.load_gather(tb_ref, [rv, fullk,
                                             jnp.full((_L,), 2, i32)])
            by1w = plsc.load_gather(tb_ref, [rv, fullk,
                                             jnp.full((_L,), 3, i32)])
            bx2w = plsc.load_gather(tb_ref, [rv, fullk,
                                             jnp.full((_L,), 4, i32)])
            by2w = plsc.load_gather(tb_ref, [rv, fullk,
                                             jnp.full((_L,), 5, i32)])
            iwi = iw.astype(i32)
            owner = (iwi >= base) & (iwi < base + _NPB)
            offw = jnp.clip(iwi - base, 0, _NPB - 1)
            clsv = plsc.load_gather(
                lg_ref, [fullk, jnp.full((_L,), offw, i32)])
            msk = (lane == 0) & owner
            tvec = jnp.full((_L,), t, i32)
            plsc.store_scatter(comb_ref.at[k], [tvec], clsv, mask=msk)
            plsc.store_scatter(comb_ref.at[4 + k], [tvec],
                               jnp.full((_L,), iw, f32), mask=msk)

            a0 = (jnp.maximum(bx2w - bx1w, 0.0) * jnp.maximum(by2w - by1w, 0.0))
            def fb(jj, mvmi, k=k, bx1w=bx1w, by1w=by1w, bx2w=bx2w, by2w=by2w,
                   a0=a0, iwi=iwi):
                mv, mi = mvmi
                for u in range(4):
                    j = jj * 4 + u
                    sl = pl.ds(j * _L, _L)
                    x = s_ref[k, sl]
                    ix1 = jnp.maximum(bx1w, bx_ref[0, sl])
                    iy1 = jnp.maximum(by1w, bx_ref[1, sl])
                    ix2 = jnp.minimum(bx2w, bx_ref[2, sl])
                    iy2 = jnp.minimum(by2w, bx_ref[3, sl])
                    inter = (jnp.maximum(ix2 - ix1, 0.0)
                             * jnp.maximum(iy2 - iy1, 0.0))
                    union = a0 + ar_ref[sl] - inter
                    iou = inter / jnp.maximum(union, jnp.float32(1e-9))
                    w = jnp.where(iou <= _IOU_THR,
                                  jnp.exp(-0.5 * iou * iou / jnp.float32(_SIGMA)),
                                  0.0)
                    gi = (base + j * _L) + lane
                    xn = jnp.where(gi == iwi, jnp.float32(-1.0), x * w)
                    s_ref[k, sl] = xn
                    cond = xn > mv
                    mv = jnp.where(cond, xn, mv)
                    mi = jnp.where(cond, gi.astype(f32), mi)
                return (mv, mi)
            mv, mi = lax.fori_loop(0, _NCH // 4, fb,
                                   (jnp.full((_L,), -3e38, f32), z16))
            m2 = jnp.max(mv)
            il2 = jnp.min(jnp.where(mv == m2, mi, f32(_BIGI)))
            _emit_row(k, m2, il2)
        pltpu.sync_copy(st_ref, table_sh.at[1 - p, tile])
        plsc.subcore_barrier()
        return carry

    lax.fori_loop(0, _T, step, 0)

    pltpu.sync_copy(comb_ref, acc_sh.at[tile])
    plsc.subcore_barrier()
    @pl.when(tile == 0)
    def _():
        def acc_body(tt, carry):
            pltpu.sync_copy(acc_sh.at[tt], fg_ref)
            for r in range(16):
                for c in range(8):
                    sl = pl.ds(c * _L, _L)
                    comb_ref[r, sl] = comb_ref[r, sl] + fg_ref[r, sl]
            return carry
        lax.fori_loop(1, _NS, acc_body, 0)
        for k in range(_KSLOT):
            pltpu.sync_copy(comb_ref.at[k], cval_out.at[cls_ids[k]])
            pltpu.sync_copy(comb_ref.at[4 + k], cidx_out.at[cls_ids[k]])


def sc_nms(s0, boxesT, area, logitT):
    f32 = jnp.float32
    return pl.kernel(
        _sc_nms_body,
        out_type=(jax.ShapeDtypeStruct((8, 128), f32),
                  jax.ShapeDtypeStruct((8, 128), f32)),
        mesh=_sc_mesh(),
        compiler_params=pltpu.CompilerParams(needs_layout_passes=False),
        scratch_types=[
            pltpu.VMEM((_KSLOT, _NPB), f32),
            pltpu.VMEM((_KSLOT, _NPB), f32),
            pltpu.VMEM((4, _NPB), f32),
            pltpu.VMEM((_NPB,), f32),
            pltpu.VMEM((16, 128), f32),
            pltpu.VMEM((_KSLOT, _L), f32),
            pltpu.VMEM((_NS, _KSLOT, _L), f32),
            pltpu.VMEM((16, 128), f32),
            pltpu.VMEM_SHARED((2, _NS, _KSLOT, _L), f32),
            pltpu.VMEM_SHARED((_NS, 16, 128), f32),
        ],
    )(s0, boxesT, area, logitT)
